# baseline (speedup 1.0000x reference)
import functools

import jax
import jax.numpy as jnp
from jax import lax
from jax.experimental import pallas as pl
from jax.experimental.pallas import tpu as pltpu
from jax.experimental.pallas import tpu_sc as plsc

N = 10000
D = 128
H = 128
E = 320000
EV = 10000
OUT = 2

NC = 2
NS = 16
NW = NC * NS
EPT = E // NW
K = 80
NCH = EPT // K
ZR = 632
ZLAST = N - (NS - 1) * ZR

EVP = 10240
KE = 80
ECH = EVP // (NW * KE)


def _tc_lin_relu(h_ref, w_ref, b_ref, o_ref):
    o_ref[...] = jnp.maximum(
        jnp.dot(h_ref[...], w_ref[...], preferred_element_type=jnp.float32)
        + b_ref[...], 0.0)


def _agg_body(p_ref, h_ref, wa_ref, wh_ref, b_ref):
    aggr = p_ref[0] + p_ref[1]
    out = (jnp.dot(aggr, wa_ref[...], preferred_element_type=jnp.float32)
           + jnp.dot(h_ref[...], wh_ref[...], preferred_element_type=jnp.float32)
           + b_ref[...])
    out = jnp.maximum(out, 0.0)
    nrm = jnp.maximum(jnp.sqrt(jnp.sum(out * out, axis=1, keepdims=True)), 1e-12)
    return out / nrm


def _tc_agg_lin(p_ref, h_ref, wa_ref, wh_ref, b_ref, lw_ref, lb_ref,
                ho_ref, to_ref):
    hn = _agg_body(p_ref, h_ref, wa_ref, wh_ref, b_ref)
    ho_ref[...] = hn
    to_ref[...] = jnp.maximum(
        jnp.dot(hn, lw_ref[...], preferred_element_type=jnp.float32)
        + lb_ref[...], 0.0)


def _tc_agg(p_ref, h_ref, wa_ref, wh_ref, b_ref, o_ref):
    o_ref[...] = _agg_body(p_ref, h_ref, wa_ref, wh_ref, b_ref)


def _tc_head(g_ref, pw1_ref, pw2_ref, pb1_ref, pb2_ref, o_ref):
    small = jnp.dot(pw1_ref[...], pw2_ref[...],
                    preferred_element_type=jnp.float32)
    c = jnp.dot(pb1_ref[...], pw2_ref[...],
                preferred_element_type=jnp.float32) + pb2_ref[...]
    z = (jnp.dot(g_ref[0], small[:H], preferred_element_type=jnp.float32)
         + jnp.dot(g_ref[1], small[H:], preferred_element_type=jnp.float32)
         + c)
    z0 = z[:, 0:1]
    z1 = z[:, 1:2]
    m = jnp.maximum(z0, z1)
    lse = m + jnp.log(jnp.exp(z0 - m) + jnp.exp(z1 - m))
    o_ref[...] = jnp.concatenate([z0 - lse, z1 - lse], axis=1)


def _sc_spmm(t_hbm, src_hbm, dst_hbm, zero_hbm, zdum_hbm, out_hbm,
             src_v, dst_v, buf0, buf1, acc_sh, gs0, gs1, ss0, ss1):
    ci = lax.axis_index("c")
    si = lax.axis_index("s")
    w = ci * NS + si
    pltpu.sync_copy(src_hbm.at[w], src_v)
    pltpu.sync_copy(dst_hbm.at[w], dst_v)

    @pl.when(si < NS - 1)
    def _():
        pltpu.sync_copy(zero_hbm, acc_sh.at[pl.ds(si * ZR, ZR)])

    @pl.when(si == NS - 1)
    def _():
        pltpu.sync_copy(zero_hbm.at[pl.ds(0, ZLAST)],
                        acc_sh.at[pl.ds((NS - 1) * ZR, ZLAST)])

    plsc.subcore_barrier()

    def wait(buf, sem):
        pltpu.make_async_copy(zdum_hbm, buf, sem).wait()

    def gath(c, buf, sem):
        pltpu.async_copy(t_hbm.at[src_v.at[pl.ds(c * K, K)]], buf, sem)

    def scat(c, buf, sem):
        pltpu.async_copy(buf, acc_sh.at[dst_v.at[c]], sem, add=True)

    gath(0, buf0, gs0)
    gath(1, buf1, gs1)
    wait(buf0, gs0)
    scat(0, buf0, ss0)
    wait(buf1, gs1)
    scat(1, buf1, ss1)

    @pl.loop(2, NCH - 1, step=2)
    def _(c):
        wait(buf0, ss0)
        gath(c, buf0, gs0)
        wait(buf1, ss1)
        gath(c + 1, buf1, gs1)
        wait(buf0, gs0)
        scat(c, buf0, ss0)
        wait(buf1, gs1)
        scat(c + 1, buf1, ss1)

    wait(buf0, ss0)
    gath(NCH - 1, buf0, gs0)
    wait(buf0, gs0)
    scat(NCH - 1, buf0, ss0)
    wait(buf0, ss0)
    wait(buf1, ss1)
    plsc.subcore_barrier()

    @pl.when(si < NS - 1)
    def _():
        pltpu.sync_copy(acc_sh.at[pl.ds(si * ZR, ZR)],
                        out_hbm.at[ci].at[pl.ds(si * ZR, ZR)])

    @pl.when(si == NS - 1)
    def _():
        pltpu.sync_copy(acc_sh.at[pl.ds((NS - 1) * ZR, ZLAST)],
                        out_hbm.at[ci].at[pl.ds((NS - 1) * ZR, ZLAST)])


def _sc_eval_gather(tab_hbm, e0_hbm, e1_hbm, out_hbm, i0_v, i1_v, bufs,
                    gsems, stsem):
    ci = lax.axis_index("c")
    si = lax.axis_index("s")
    w = ci * NS + si
    pltpu.sync_copy(e0_hbm.at[w], i0_v)
    pltpu.sync_copy(e1_hbm.at[w], i1_v)

    descs = []
    for c in range(ECH):
        descs.append(pltpu.async_copy(
            tab_hbm.at[i0_v.at[pl.ds(c * KE, KE)]], bufs[2 * c], gsems[2 * c]))
        descs.append(pltpu.async_copy(
            tab_hbm.at[i1_v.at[pl.ds(c * KE, KE)]], bufs[2 * c + 1],
            gsems[2 * c + 1]))
    for c in range(ECH):
        base = w * (ECH * KE) + c * KE
        descs[2 * c].wait()
        pltpu.sync_copy(bufs[2 * c], out_hbm.at[0].at[pl.ds(base, KE)])
        descs[2 * c + 1].wait()
        pltpu.sync_copy(bufs[2 * c + 1], out_hbm.at[1].at[pl.ds(base, KE)])
    del stsem


def _vmesh():
    return plsc.VectorSubcoreMesh(core_axis_name="c", subcore_axis_name="s")


def kernel(x, edge_index, batch, eval_edges, lin_W0, lin_b0, agg_W0, agg_b0,
           lin_W1, lin_b1, agg_W1, agg_b1, pW1, pb1, pW2, pb2):
    del batch

    f32 = jnp.float32
    src = edge_index[0].reshape(NW, EPT)
    dst = edge_index[1].reshape(NW, NCH, K)
    zero_rows = jnp.zeros((ZR, H), f32)
    zdum = jnp.zeros((K, H), f32)

    pad = jnp.zeros((EVP - EV,), jnp.int32)
    e0 = jnp.concatenate([eval_edges[0], pad]).reshape(NW, ECH * KE)
    e1 = jnp.concatenate([eval_edges[1], pad]).reshape(NW, ECH * KE)

    lin_relu = pl.pallas_call(
        _tc_lin_relu, out_shape=jax.ShapeDtypeStruct((N, H), f32))
    agg_lin = pl.pallas_call(
        _tc_agg_lin, out_shape=(jax.ShapeDtypeStruct((N, H), f32),
                                jax.ShapeDtypeStruct((N, H), f32)))
    agg = pl.pallas_call(
        _tc_agg, out_shape=jax.ShapeDtypeStruct((N, H), f32))
    head = pl.pallas_call(
        _tc_head, out_shape=jax.ShapeDtypeStruct((EVP, OUT), f32))

    spmm = functools.partial(
        pl.kernel,
        out_type=jax.ShapeDtypeStruct((NC, N, H), f32),
        mesh=_vmesh(),
        scratch_types=[
            pltpu.VMEM((EPT,), jnp.int32),
            pltpu.VMEM((NCH, K), jnp.int32),
            pltpu.VMEM((K, H), f32),
            pltpu.VMEM((K, H), f32),
            pltpu.VMEM_SHARED((N, H), f32),
            pltpu.SemaphoreType.DMA,
            pltpu.SemaphoreType.DMA,
            pltpu.SemaphoreType.DMA,
            pltpu.SemaphoreType.DMA,
        ],
    )(_sc_spmm)

    def eval_body(tab_hbm, e0_hbm, e1_hbm, out_hbm, i0_v, i1_v,
                  b0, b1, b2, b3, b4, b5, b6, b7,
                  g0, g1, g2, g3, g4, g5, g6, g7, stsem):
        _sc_eval_gather(tab_hbm, e0_hbm, e1_hbm, out_hbm, i0_v, i1_v,
                        [b0, b1, b2, b3, b4, b5, b6, b7],
                        [g0, g1, g2, g3, g4, g5, g6, g7], stsem)

    eval_gather = functools.partial(
        pl.kernel,
        out_type=jax.ShapeDtypeStruct((2, EVP, H), f32),
        mesh=_vmesh(),
        scratch_types=(
            [pltpu.VMEM((ECH * KE,), jnp.int32)] * 2
            + [pltpu.VMEM((KE, H), f32)] * 8
            + [pltpu.SemaphoreType.DMA] * 9
        ),
    )(eval_body)

    t = lin_relu(x, lin_W0, lin_b0.reshape(1, H))
    parts = spmm(t, src, dst, zero_rows, zdum)
    h1, t1 = agg_lin(parts, x, agg_W0[:H], agg_W0[H:], agg_b0.reshape(1, H),
                     lin_W1, lin_b1.reshape(1, H))
    parts = spmm(t1, src, dst, zero_rows, zdum)
    h2 = agg(parts, h1, agg_W1[:H], agg_W1[H:], agg_b1.reshape(1, H))

    g = eval_gather(h2, e0, e1)
    out = head(g, pW1, pW2, pb1.reshape(1, H), pb2.reshape(1, OUT))
    return out[:EV]


# speedup vs baseline: 1.2799x; 1.2799x over previous
import functools

import jax
import jax.numpy as jnp
from jax import lax
from jax.experimental import pallas as pl
from jax.experimental.pallas import tpu as pltpu
from jax.experimental.pallas import tpu_sc as plsc

N = 10000
D = 128
H = 128
E = 320000
EV = 10000
OUT = 2

NC = 2
NS = 16
NW = NC * NS
EPT = E // NW
K = 128
NFULL = EPT // K
KT = EPT - NFULL * K
DHALF = 40
ZR = 632
ZLAST = N - (NS - 1) * ZR

EVP = 10240
KE = 80
ECH = EVP // (NW * KE)


def _tc_lin_relu(h_ref, w_ref, b_ref, o_ref):
    o_ref[...] = jnp.maximum(
        jnp.dot(h_ref[...], w_ref[...], preferred_element_type=jnp.float32)
        + b_ref[...], 0.0)


def _agg_body(p_ref, h_ref, wa_ref, wh_ref, b_ref):
    aggr = p_ref[0] + p_ref[1]
    out = (jnp.dot(aggr, wa_ref[...], preferred_element_type=jnp.float32)
           + jnp.dot(h_ref[...], wh_ref[...], preferred_element_type=jnp.float32)
           + b_ref[...])
    out = jnp.maximum(out, 0.0)
    nrm = jnp.maximum(jnp.sqrt(jnp.sum(out * out, axis=1, keepdims=True)), 1e-12)
    return out / nrm


def _tc_agg_lin(p_ref, h_ref, wa_ref, wh_ref, b_ref, lw_ref, lb_ref,
                ho_ref, to_ref):
    hn = _agg_body(p_ref, h_ref, wa_ref, wh_ref, b_ref)
    ho_ref[...] = hn
    to_ref[...] = jnp.maximum(
        jnp.dot(hn, lw_ref[...], preferred_element_type=jnp.float32)
        + lb_ref[...], 0.0)


def _tc_agg(p_ref, h_ref, wa_ref, wh_ref, b_ref, o_ref):
    o_ref[...] = _agg_body(p_ref, h_ref, wa_ref, wh_ref, b_ref)


def _tc_head(g_ref, pw1_ref, pw2_ref, pb1_ref, pb2_ref, o_ref):
    small = jnp.dot(pw1_ref[...], pw2_ref[...],
                    preferred_element_type=jnp.float32)
    c = jnp.dot(pb1_ref[...], pw2_ref[...],
                preferred_element_type=jnp.float32) + pb2_ref[...]
    z = (jnp.dot(g_ref[0], small[:H], preferred_element_type=jnp.float32)
         + jnp.dot(g_ref[1], small[H:], preferred_element_type=jnp.float32)
         + c)
    z0 = z[:, 0:1]
    z1 = z[:, 1:2]
    m = jnp.maximum(z0, z1)
    lse = m + jnp.log(jnp.exp(z0 - m) + jnp.exp(z1 - m))
    o_ref[...] = jnp.concatenate([z0 - lse, z1 - lse], axis=1)


def _sc_spmm(t_hbm, src_hbm, dst_hbm, dtail_hbm, zero_hbm, zdum_hbm, out_hbm,
             src_v, dst_v, dtail_v, buf0, buf1, acc_sh, gs0, gs1):
    ci = lax.axis_index("c")
    si = lax.axis_index("s")
    w = ci * NS + si
    pltpu.sync_copy(src_hbm.at[w], src_v)
    pltpu.sync_copy(dst_hbm.at[w].at[pl.ds(0, DHALF)], dst_v)
    pltpu.sync_copy(dtail_hbm.at[w], dtail_v)

    @pl.when(si < NS - 1)
    def _():
        pltpu.sync_copy(zero_hbm, acc_sh.at[pl.ds(si * ZR, ZR)])

    @pl.when(si == NS - 1)
    def _():
        pltpu.sync_copy(zero_hbm.at[pl.ds(0, ZLAST)],
                        acc_sh.at[pl.ds((NS - 1) * ZR, ZLAST)])

    plsc.subcore_barrier()

    def wait(buf, sem):
        pltpu.make_async_copy(zdum_hbm, buf, sem).wait()

    def gath(c, buf, sem):
        pltpu.async_copy(t_hbm.at[src_v.at[pl.ds(c * K, K)]], buf, sem)

    gath(0, buf0, gs0)

    @pl.loop(0, NFULL, step=2)
    def _(c):
        @pl.when(c == DHALF)
        def _():
            pltpu.sync_copy(dst_hbm.at[w].at[pl.ds(DHALF, DHALF)], dst_v)

        r0 = jnp.where(c >= DHALF, c - DHALF, c)
        gath(c + 1, buf1, gs1)
        wait(buf0, gs0)
        pltpu.sync_copy(buf0, acc_sh.at[dst_v.at[r0]], add=True)

        @pl.when(c + 2 < NFULL)
        def _():
            gath(c + 2, buf0, gs0)

        wait(buf1, gs1)
        pltpu.sync_copy(buf1, acc_sh.at[dst_v.at[r0 + 1]], add=True)

    pltpu.async_copy(t_hbm.at[src_v.at[pl.ds(NFULL * K, KT)]],
                     buf0.at[pl.ds(0, KT)], gs0)
    pltpu.make_async_copy(zdum_hbm.at[pl.ds(0, KT)], buf0.at[pl.ds(0, KT)],
                          gs0).wait()
    pltpu.sync_copy(buf0.at[pl.ds(0, KT)], acc_sh.at[dtail_v.at[0]], add=True)
    plsc.subcore_barrier()

    @pl.when(si < NS - 1)
    def _():
        pltpu.sync_copy(acc_sh.at[pl.ds(si * ZR, ZR)],
                        out_hbm.at[ci].at[pl.ds(si * ZR, ZR)])

    @pl.when(si == NS - 1)
    def _():
        pltpu.sync_copy(acc_sh.at[pl.ds((NS - 1) * ZR, ZLAST)],
                        out_hbm.at[ci].at[pl.ds((NS - 1) * ZR, ZLAST)])


def _sc_eval_gather(tab_hbm, e0_hbm, e1_hbm, out_hbm, i0_v, i1_v, bufs,
                    gsems, stsem):
    ci = lax.axis_index("c")
    si = lax.axis_index("s")
    w = ci * NS + si
    pltpu.sync_copy(e0_hbm.at[w], i0_v)
    pltpu.sync_copy(e1_hbm.at[w], i1_v)

    descs = []
    for c in range(ECH):
        descs.append(pltpu.async_copy(
            tab_hbm.at[i0_v.at[pl.ds(c * KE, KE)]], bufs[2 * c], gsems[2 * c]))
        descs.append(pltpu.async_copy(
            tab_hbm.at[i1_v.at[pl.ds(c * KE, KE)]], bufs[2 * c + 1],
            gsems[2 * c + 1]))
    for c in range(ECH):
        base = w * (ECH * KE) + c * KE
        descs[2 * c].wait()
        pltpu.sync_copy(bufs[2 * c], out_hbm.at[0].at[pl.ds(base, KE)])
        descs[2 * c + 1].wait()
        pltpu.sync_copy(bufs[2 * c + 1], out_hbm.at[1].at[pl.ds(base, KE)])
    del stsem


def _vmesh():
    return plsc.VectorSubcoreMesh(core_axis_name="c", subcore_axis_name="s")


def kernel(x, edge_index, batch, eval_edges, lin_W0, lin_b0, agg_W0, agg_b0,
           lin_W1, lin_b1, agg_W1, agg_b1, pW1, pb1, pW2, pb2):
    del batch

    f32 = jnp.float32
    src = edge_index[0].reshape(NW, EPT)
    d = edge_index[1].reshape(NW, EPT)
    dst = (jnp.zeros((NW, 2 * DHALF, K), jnp.int32)
           .at[:, :NFULL].set(d[:, :NFULL * K].reshape(NW, NFULL, K)))
    dtail = d[:, NFULL * K:].reshape(NW, 1, KT)
    zero_rows = jnp.zeros((ZR, H), f32)
    zdum = jnp.zeros((K, H), f32)

    pad = jnp.zeros((EVP - EV,), jnp.int32)
    e0 = jnp.concatenate([eval_edges[0], pad]).reshape(NW, ECH * KE)
    e1 = jnp.concatenate([eval_edges[1], pad]).reshape(NW, ECH * KE)

    lin_relu = pl.pallas_call(
        _tc_lin_relu, out_shape=jax.ShapeDtypeStruct((N, H), f32))
    agg_lin = pl.pallas_call(
        _tc_agg_lin, out_shape=(jax.ShapeDtypeStruct((N, H), f32),
                                jax.ShapeDtypeStruct((N, H), f32)))
    agg = pl.pallas_call(
        _tc_agg, out_shape=jax.ShapeDtypeStruct((N, H), f32))
    head = pl.pallas_call(
        _tc_head, out_shape=jax.ShapeDtypeStruct((EVP, OUT), f32))

    spmm = functools.partial(
        pl.kernel,
        out_type=jax.ShapeDtypeStruct((NC, N, H), f32),
        mesh=_vmesh(),
        scratch_types=[
            pltpu.VMEM((EPT,), jnp.int32),
            pltpu.VMEM((DHALF, K), jnp.int32),
            pltpu.VMEM((1, KT), jnp.int32),
            pltpu.VMEM((K, H), f32),
            pltpu.VMEM((K, H), f32),
            pltpu.VMEM_SHARED((N, H), f32),
            pltpu.SemaphoreType.DMA,
            pltpu.SemaphoreType.DMA,
        ],
    )(_sc_spmm)

    def eval_body(tab_hbm, e0_hbm, e1_hbm, out_hbm, i0_v, i1_v,
                  b0, b1, b2, b3, b4, b5, b6, b7,
                  g0, g1, g2, g3, g4, g5, g6, g7, stsem):
        _sc_eval_gather(tab_hbm, e0_hbm, e1_hbm, out_hbm, i0_v, i1_v,
                        [b0, b1, b2, b3, b4, b5, b6, b7],
                        [g0, g1, g2, g3, g4, g5, g6, g7], stsem)

    eval_gather = functools.partial(
        pl.kernel,
        out_type=jax.ShapeDtypeStruct((2, EVP, H), f32),
        mesh=_vmesh(),
        scratch_types=(
            [pltpu.VMEM((ECH * KE,), jnp.int32)] * 2
            + [pltpu.VMEM((KE, H), f32)] * 8
            + [pltpu.SemaphoreType.DMA] * 9
        ),
    )(eval_body)

    t = lin_relu(x, lin_W0, lin_b0.reshape(1, H))
    parts = spmm(t, src, dst, dtail, zero_rows, zdum)
    h1, t1 = agg_lin(parts, x, agg_W0[:H], agg_W0[H:], agg_b0.reshape(1, H),
                     lin_W1, lin_b1.reshape(1, H))
    parts = spmm(t1, src, dst, dtail, zero_rows, zdum)
    h2 = agg(parts, h1, agg_W1[:H], agg_W1[H:], agg_b1.reshape(1, H))

    g = eval_gather(h2, e0, e1)
    out = head(g, pW1, pW2, pb1.reshape(1, H), pb2.reshape(1, OUT))
    return out[:EV]
